# Initial kernel scaffold; baseline (speedup 1.0000x reference)
import jax
import jax.numpy as jnp
from jax.experimental import pallas as pl


def kernel(x, table):
    raise NotImplementedError("write your pallas kernel here")


# speedup vs baseline: 20.6343x; 20.6343x over previous
import jax
import jax.numpy as jnp
import numpy as np
from jax import lax
from jax.experimental import pallas as pl
from jax.experimental.pallas import tpu as pltpu
from jax.experimental.pallas import tpu_sc as plsc

N_LEVELS = 16
F = 2
LOG2_T = 19
T = 1 << LOG2_T
BASE_RES = 16
PER_LEVEL_SCALE = 1.3819
N_POINTS = 262144
P1 = 2654435761
P2 = 805459861

NC, NS, LANES = 2, 16, 16
NW = NC * NS
PPT = N_POINTS // NW
B = 512
NBLK = PPT // B
NIT = B // LANES
OUTW = N_LEVELS * F

_RES = [int(np.floor(BASE_RES * PER_LEVEL_SCALE ** l)) for l in range(N_LEVELS)]
_DENSE = [(r + 1) ** 3 <= T for r in _RES]
_CORNERS = [(i, j, k) for i in range(2) for j in range(2) for k in range(2)]


def _body(x0_hbm, x1_hbm, x2_hbm, tab_hbm, out_hbm,
          x0v, x1v, x2v,
          idx0a, idx1a, dst0a, dst1a,
          idx0b, idx1b, dst0b, dst1b,
          outv, sema, semb):
    wid = lax.axis_index("s") * NC + lax.axis_index("c")
    base = wid * PPT
    pltpu.sync_copy(x0_hbm.at[pl.ds(base, PPT)], x0v)
    pltpu.sync_copy(x1_hbm.at[pl.ds(base, PPT)], x1v)
    pltpu.sync_copy(x2_hbm.at[pl.ds(base, PPT)], x2v)

    iota = lax.iota(jnp.int32, LANES)
    bufs = ((idx0a, idx1a, dst0a, dst1a, sema),
            (idx0b, idx1b, dst0b, dst1b, semb))

    def emit_a(lvl, bb, i0v, i1v):
        def pass_a(i, c_):
            s = bb + i * LANES
            resf = jnp.float32(_RES[lvl])
            p0 = x0v[pl.ds(s, LANES)] * resf
            p1 = x1v[pl.ds(s, LANES)] * resf
            p2 = x2v[pl.ds(s, LANES)] * resf
            woff = 2 * lvl * T
            if _DENSE[lvl]:
                n1 = _RES[lvl] + 1
                n2 = n1 * n1
                c0 = p0.astype(jnp.int32)
                c1 = p1.astype(jnp.int32)
                c2 = p2.astype(jnp.int32)
                bidx = c0 * (2 * n2) + c1 * (2 * n1) + c2 * 2 + woff
                for c, (ci, cj, ck) in enumerate(_CORNERS):
                    w0 = bidx + (2 * (ci * n2 + cj * n1 + ck))
                    i0v[pl.ds(c * B + i * LANES, LANES)] = w0
                    i1v[pl.ds(c * B + i * LANES, LANES)] = w0 + 1
            else:
                u0 = p0.astype(jnp.uint32)
                u1 = p1.astype(jnp.uint32)
                u2 = p2.astype(jnp.uint32)
                h1 = u1 * jnp.uint32(P1)
                h2 = u2 * jnp.uint32(P2)
                a = (u0, u0 + jnp.uint32(1))
                b = (h1, h1 + jnp.uint32(P1))
                d = (h2, h2 + jnp.uint32(P2))
                for c, (ci, cj, ck) in enumerate(_CORNERS):
                    hv = (a[ci] ^ b[cj] ^ d[ck]) & jnp.uint32(T - 1)
                    w0 = hv.astype(jnp.int32) * 2 + woff
                    i0v[pl.ds(c * B + i * LANES, LANES)] = w0
                    i1v[pl.ds(c * B + i * LANES, LANES)] = w0 + 1
            return c_

        lax.fori_loop(0, NIT, pass_a, 0)

    def emit_b(lvl, bb, d0v, d1v):
        def pass_b(i, c_):
            s = bb + i * LANES
            resf = jnp.float32(_RES[lvl])
            p0 = x0v[pl.ds(s, LANES)] * resf
            p1 = x1v[pl.ds(s, LANES)] * resf
            p2 = x2v[pl.ds(s, LANES)] * resf
            f0 = p0 - p0.astype(jnp.int32).astype(jnp.float32)
            f1 = p1 - p1.astype(jnp.int32).astype(jnp.float32)
            f2 = p2 - p2.astype(jnp.int32).astype(jnp.float32)
            m0 = 1.0 - f0
            m1 = 1.0 - f1
            m2 = 1.0 - f2
            acc0 = jnp.zeros((LANES,), jnp.float32)
            acc1 = jnp.zeros((LANES,), jnp.float32)
            for c, (ci, cj, ck) in enumerate(_CORNERS):
                w = ((f0 if ci else m0) * (f1 if cj else m1)
                     * (f2 if ck else m2))
                g0 = d0v[pl.ds(c * B + i * LANES, LANES)]
                g1 = d1v[pl.ds(c * B + i * LANES, LANES)]
                acc0 = acc0 + w * g0
                acc1 = acc1 + w * g1
            w0 = (i * LANES + iota) * OUTW + (2 * lvl)
            plsc.store_scatter(outv, [w0], acc0)
            plsc.store_scatter(outv, [w0 + 1], acc1)
            return c_

        lax.fori_loop(0, NIT, pass_b, 0)

    def fire(i0v, i1v, d0v, d1v, sem):
        c0 = pltpu.async_copy(tab_hbm.at[i0v], d0v, sem)
        c1 = pltpu.async_copy(tab_hbm.at[i1v], d1v, sem)
        return (c0, c1)

    def block(blk, carry):
        bb = blk * B

        emit_a(0, bb, *bufs[0][:2])
        inflight = fire(*bufs[0])
        for l in range(N_LEVELS):
            cur, nxt = bufs[l % 2], bufs[(l + 1) % 2]
            if l + 1 < N_LEVELS:
                emit_a(l + 1, bb, *nxt[:2])
                nxt_inflight = fire(*nxt)
            for cp in inflight:
                cp.wait()
            emit_b(l, bb, *cur[2:4])
            if l + 1 < N_LEVELS:
                inflight = nxt_inflight

        pltpu.sync_copy(outv, out_hbm.at[pl.ds((base + bb) * OUTW, B * OUTW)])
        return carry

    lax.fori_loop(0, NBLK, block, 0)


@jax.jit
def _encode(x0, x1, x2, tab):
    mesh = plsc.VectorSubcoreMesh(core_axis_name="c", subcore_axis_name="s")
    fn = pl.kernel(
        _body,
        out_type=jax.ShapeDtypeStruct((N_POINTS * OUTW,), jnp.float32),
        mesh=mesh,
        compiler_params=pltpu.CompilerParams(needs_layout_passes=False),
        scratch_types=[
            pltpu.VMEM((PPT,), jnp.float32),
            pltpu.VMEM((PPT,), jnp.float32),
            pltpu.VMEM((PPT,), jnp.float32),
            pltpu.VMEM((B * 8,), jnp.int32),
            pltpu.VMEM((B * 8,), jnp.int32),
            pltpu.VMEM((B * 8,), jnp.float32),
            pltpu.VMEM((B * 8,), jnp.float32),
            pltpu.VMEM((B * 8,), jnp.int32),
            pltpu.VMEM((B * 8,), jnp.int32),
            pltpu.VMEM((B * 8,), jnp.float32),
            pltpu.VMEM((B * 8,), jnp.float32),
            pltpu.VMEM((B * OUTW,), jnp.float32),
            pltpu.SemaphoreType.DMA,
            pltpu.SemaphoreType.DMA,
        ],
    )
    return fn(x0, x1, x2, tab)


def kernel(x, table):
    xt = x.T
    tab = table.reshape(-1)
    out = _encode(xt[0], xt[1], xt[2], tab)
    return out.reshape(N_POINTS, OUTW)
